# baseline (speedup 1.0000x reference)
import functools

import jax
import jax.numpy as jnp
from jax import lax
from jax.experimental import pallas as pl
from jax.experimental.pallas import tpu as pltpu
from jax.experimental.pallas import tpu_sc as plsc

N = 10000
E = 320000
F = 128
FH = 64
NLAYERS = 2
NG = 64
BN_EPS = 1e-5
LN_EPS = 1e-5

NC, NS = 2, 16
E2 = 327680
EPT = E2 // NS
B = 40
NB = EPT // B
MB = 32
NSB = NB // MB
NPA = 10008
TRASH = 10000
EROW = B * FH // 512

NT = 1000
EPACK = E2 // 8
ETP = 2048


def _data_body(x_ref, w_ref, b_ref, full_ref):
    res = jnp.dot(x_ref[...], w_ref[0], preferred_element_type=jnp.float32)
    full_ref[0] = res + b_ref[0, 0]


def _node_encode(x, ne_W, ne_b):
    return pl.pallas_call(
        _data_body,
        grid=(NLAYERS, N // NT),
        in_specs=[
            pl.BlockSpec((NT, F), lambda l, i: (i, 0)),
            pl.BlockSpec((1, F, F), lambda l, i: (l, 0, 0)),
            pl.BlockSpec((1, 1, F), lambda l, i: (l, 0, 0)),
        ],
        out_specs=pl.BlockSpec((1, NT, F), lambda l, i: (l, i, 0)),
        out_shape=jax.ShapeDtypeStruct((NLAYERS, N, F), jnp.float32),
    )(x, ne_W, ne_b)


def _ea_body(e_ref, w_ref, b_ref, o_ref):
    o_ref[0] = (
        jnp.dot(e_ref[...], w_ref[0], preferred_element_type=jnp.float32)
        + b_ref[0, 0]
    )


def _edge_encode(edge_attr2, wb, bb):
    return pl.pallas_call(
        _ea_body,
        grid=(2, EPACK // ETP),
        in_specs=[
            pl.BlockSpec((ETP, F), lambda c, i: (i, 0)),
            pl.BlockSpec((1, F, 8 * FH), lambda c, i: (c, 0, 0)),
            pl.BlockSpec((1, 1, 8 * FH), lambda c, i: (c, 0, 0)),
        ],
        out_specs=pl.BlockSpec((1, ETP, 8 * FH), lambda c, i: (c, i, 0)),
        out_shape=jax.ShapeDtypeStruct((2, EPACK, 8 * FH), jnp.float32),
    )(edge_attr2, wb, bb)


_OWN = 624
_CHUNKS = [(k * B, B) for k in range(15)] + [(15 * B, 24)]
_XTRA = (_OWN, 24)


def _sc_body(data_hbm, src_hbm, dst_hbm, ea_hbm, out_hbm,
             acc_sh, src2, dst2, dst_w, rows_v0, rows_v1, ea_v0, ea_v1,
             ctr_v, fin_v, semg, seme, semsc):
    c = lax.axis_index("c")
    s = lax.axis_index("s")
    row0 = s * _OWN
    e0t = s * EPT
    col0 = c * FH

    def _zb(i, _):
        ctr_v[i // 8, pl.ds((i % 8) * 16, 16)] = jnp.zeros((16,), jnp.float32)
        return _
    lax.fori_loop(0, B * 8, _zb, None)
    for off, size in _CHUNKS:
        pltpu.sync_copy(ctr_v.at[pl.ds(0, size), :],
                        acc_sh.at[pl.ds(row0 + off, size), :])

    @pl.when(s == NS - 1)
    def _():
        pltpu.sync_copy(ctr_v.at[pl.ds(0, _XTRA[1]), :],
                        acc_sh.at[pl.ds(row0 + _XTRA[0], _XTRA[1]), :])
    plsc.subcore_barrier()

    def _issue(j, mrow0, rbank, ebank):
        pltpu.async_copy(
            data_hbm.at[src2.at[pl.ds(j * B, B)]], rbank, semg)
        pltpu.async_copy(
            ea_hbm.at[c, pl.ds(mrow0 + j * B, B), :], ebank, seme)

    def _mb(j, rbank, ebank):
        pltpu.make_async_copy(
            data_hbm.at[pl.ds(0, B), :], rbank, semg).wait()
        pltpu.make_async_copy(
            ea_hbm.at[c, pl.ds(0, B), :], ebank, seme).wait()

        @plsc.parallel_loop(0, B, unroll=2)
        def _cb(e):
            for v in range(FH // 16):
                a = ebank[e, pl.ds(v * 16, 16)]
                g = rbank[e, pl.ds(col0 + v * 16, 16)]
                m = jnp.maximum(g + a, 0.0) + 1e-7
                ex = jnp.exp(m)
                ctr_v[e, pl.ds(v * 16, 16)] = m * ex
                ctr_v[e, pl.ds(FH + v * 16, 16)] = ex

        for o in (0, 16, B - 16):
            dst_w[pl.ds(o, 16)] = dst2[pl.ds(j * B + o, 16)]
        pltpu.sync_copy(ctr_v, acc_sh.at[dst_w], add=True)

    def _sb(sb, _):
        mrow0 = e0t + sb * MB * B
        pltpu.sync_copy(src_hbm.at[pl.ds(mrow0, MB * B)], src2)
        pltpu.sync_copy(dst_hbm.at[pl.ds(mrow0, MB * B)], dst2)
        _issue(0, mrow0, rows_v0, ea_v0)

        def _pair(k, _2):
            j0 = 2 * k
            _issue(j0 + 1, mrow0, rows_v1, ea_v1)
            _mb(j0, rows_v0, ea_v0)

            @pl.when(k < MB // 2 - 1)
            def _():
                _issue(j0 + 2, mrow0, rows_v0, ea_v0)
            _mb(j0 + 1, rows_v1, ea_v1)
            return _2
        lax.fori_loop(0, MB // 2, _pair, None)
        return _
    lax.fori_loop(0, NSB, _sb, None)

    plsc.subcore_barrier()

    def _fin(off, size):
        r0 = row0 + off
        pltpu.sync_copy(acc_sh.at[pl.ds(r0, size), :],
                        ctr_v.at[pl.ds(0, size), :])

        def _fb(i, _):
            for v in range(FH // 16):
                num = ctr_v[i, pl.ds(v * 16, 16)]
                den = ctr_v[i, pl.ds(FH + v * 16, 16)]
                fin_v[i, pl.ds(v * 16, 16)] = num / (den + 1e-16)
            return _
        lax.fori_loop(0, size, _fb, None)
        pltpu.sync_copy(fin_v.at[pl.ds(0, size), :],
                        out_hbm.at[c, pl.ds(r0, size), :])

    for off, size in _CHUNKS:
        _fin(off, size)

    @pl.when(s == NS - 1)
    def _():
        _fin(*_XTRA)


@functools.cache
def _make_sc_aggregate():
    return functools.partial(
        pl.kernel,
        out_type=jax.ShapeDtypeStruct((2, NPA, FH), jnp.float32),
        mesh=plsc.VectorSubcoreMesh(core_axis_name="c", subcore_axis_name="s",
                                    num_cores=NC, num_subcores=NS),
        scratch_types=[
            pltpu.VMEM_SHARED((NPA, 2 * FH), jnp.float32),
            pltpu.VMEM((MB * B,), jnp.int32),
            pltpu.VMEM((MB * B,), jnp.int32),
            pltpu.VMEM((B,), jnp.int32),
            pltpu.VMEM((B, F), jnp.float32),
            pltpu.VMEM((B, F), jnp.float32),
            pltpu.VMEM((B, FH), jnp.float32),
            pltpu.VMEM((B, FH), jnp.float32),
            pltpu.VMEM((B, 2 * FH), jnp.float32),
            pltpu.VMEM((B, FH), jnp.float32),
            pltpu.SemaphoreType.DMA,
            pltpu.SemaphoreType.DMA,
            pltpu.SemaphoreType.DMA,
        ],
    )(_sc_body)


def _node_body(agg0_ref, agg1_ref, data_ref, scale_ref, w1_ref, b1_ref,
               w2_ref, b2_ref, lng_ref, lnb_ref, batch_ref,
               cw0_ref, cb0_ref, cw1_ref, cb1_ref, cw2_ref, cb2_ref,
               cw3_ref, cb3_ref, o_ref, pooled):
    i = pl.program_id(0)
    nsteps = pl.num_programs(0)

    @pl.when(i == 0)
    def _():
        pooled[...] = jnp.zeros_like(pooled)

    r = jnp.zeros((NT, F), jnp.float32)
    for l, agg_ref in ((0, agg0_ref), (1, agg1_ref)):
        a = jnp.concatenate([agg_ref[0], agg_ref[1]], axis=1)
        d = data_ref[l]
        nrm2 = jnp.sqrt(jnp.sum(a * a, axis=1, keepdims=True))
        msgn = a / jnp.maximum(nrm2, 1e-12)
        xn = jnp.sqrt(jnp.sum(d * d, axis=1, keepdims=True))
        out = msgn * xn * scale_ref[l, 0] + d
        h = jnp.dot(out, w1_ref[l], preferred_element_type=jnp.float32)
        h = jnp.maximum(h + b1_ref[l, 0], 0.0)
        h = jnp.dot(h, w2_ref[l], preferred_element_type=jnp.float32)
        h = h + b2_ref[l, 0]
        mu = jnp.mean(h, axis=1, keepdims=True)
        var = jnp.mean((h - mu) ** 2, axis=1, keepdims=True)
        h = (h - mu) / jnp.sqrt(var + LN_EPS) * lng_ref[l, 0] + lnb_ref[l, 0]
        h = jnp.maximum(h, 0.0)
        hmax = jnp.max(h, axis=1, keepdims=True)
        eh = jnp.exp(h - hmax)
        r = r + eh / jnp.sum(eh, axis=1, keepdims=True)

    bt = batch_ref[0, 0]
    gid = jax.lax.broadcasted_iota(jnp.int32, (NT, NG), 1)
    onehot = jnp.where(bt[:, None] == gid, 1.0, 0.0).astype(jnp.float32)
    pooled[...] += jax.lax.dot_general(
        onehot, r, (((0,), (0,)), ((), ())),
        preferred_element_type=jnp.float32)

    @pl.when(i == nsteps - 1)
    def _():
        g = pooled[...]
        g = jnp.maximum(
            jnp.dot(g, cw0_ref[...], preferred_element_type=jnp.float32)
            + cb0_ref[0], 0.0)
        g = jnp.maximum(
            jnp.dot(g, cw1_ref[...], preferred_element_type=jnp.float32)
            + cb1_ref[0], 0.0)
        g = jnp.maximum(
            jnp.dot(g, cw2_ref[...], preferred_element_type=jnp.float32)
            + cb2_ref[0], 0.0)
        g = jnp.dot(g, cw3_ref[...], preferred_element_type=jnp.float32)
        g = g + cb3_ref[0]
        o_ref[...] = jax.nn.sigmoid(g)


def _node_stage(agg0, agg1, data_full, scale, w1, b1, w2, b2, lng, lnb,
                batch3, cls_w, cls_b):
    full = lambda shape: pl.BlockSpec(shape, lambda i: tuple(0 for _ in shape))
    return pl.pallas_call(
        _node_body,
        grid=(N // NT,),
        in_specs=[
            pl.BlockSpec((2, NT, FH), lambda i: (0, i, 0)),
            pl.BlockSpec((2, NT, FH), lambda i: (0, i, 0)),
            pl.BlockSpec((NLAYERS, NT, F), lambda i: (0, i, 0)),
            full((NLAYERS, 1, F)),
            full((NLAYERS, F, 2 * F)),
            full((NLAYERS, 1, 2 * F)),
            full((NLAYERS, 2 * F, F)),
            full((NLAYERS, 1, F)),
            full((NLAYERS, 1, F)),
            full((NLAYERS, 1, F)),
            pl.BlockSpec((1, 1, NT), lambda i: (i, 0, 0)),
            full((F, 2 * F)),
            full((1, 2 * F)),
            full((2 * F, F)),
            full((1, F)),
            full((F, NG)),
            full((1, NG)),
            full((NG, 1)),
            full((1, 1)),
        ],
        out_specs=pl.BlockSpec((NG, 1), lambda i: (0, 0)),
        out_shape=jax.ShapeDtypeStruct((NG, 1), jnp.float32),
        scratch_shapes=[pltpu.VMEM((NG, F), jnp.float32)],
    )(agg0, agg1, data_full, scale, w1, b1, w2, b2, lng, lnb, batch3,
      *[a for pair in zip(cls_w, cls_b) for a in pair])


def kernel(x, edge_attr, params, edge_index, batch):
    layers = params['layers']
    cls = params['classifier']

    ne_W = jnp.stack([p['ne_W'] for p in layers])
    ne_b = jnp.stack([p['ne_b'] for p in layers])[:, None, :]

    def _blockdiag(p):
        halves = []
        for c2 in range(2):
            h = p['ee_W'][:, c2 * FH:(c2 + 1) * FH]
            halves.append(jax.scipy.linalg.block_diag(*([h] * 8)))
        return jnp.stack(halves)

    wb = [_blockdiag(p) for p in layers]
    bb = [jnp.stack([jnp.tile(p['ee_b'][c2 * FH:(c2 + 1) * FH], 8)[None, :]
                     for c2 in range(2)]) for p in layers]

    sbn = 1.0 / jnp.sqrt(1.0 + BN_EPS)
    w1 = jnp.stack([p['mlp_W1'] * (sbn * p['mlp_bn_g'])[None, :]
                    for p in layers])
    b1 = jnp.stack([(p['mlp_b1'] * sbn * p['mlp_bn_g'] + p['mlp_bn_b'])
                    for p in layers])[:, None, :]
    w2 = jnp.stack([p['mlp_W2'] for p in layers])
    b2 = jnp.stack([p['mlp_b2'] for p in layers])[:, None, :]
    lng = jnp.stack([p['ln_g'] for p in layers])[:, None, :]
    lnb = jnp.stack([p['ln_b'] for p in layers])[:, None, :]
    scale = jnp.stack([jnp.broadcast_to(p['msg_scale'], (F,))
                       for p in layers])[:, None, :]

    cls_w, cls_b = [], []
    cur_s, cur_t = None, None
    for ci, c in enumerate(cls):
        W, b = c['W'], c['b']
        if cur_s is not None:
            W = cur_s[:, None] * W
            b = b + cur_t @ c['W']
        cls_w.append(W)
        cls_b.append(b[None, :])
        if ci < 3:
            cur_s = sbn * c['bn_g']
            cur_t = c['bn_b']

    src = jnp.pad(edge_index[0], (0, E2 - E))
    dst = jnp.pad(edge_index[1], (0, E2 - E), constant_values=TRASH)
    batch3 = batch.reshape(N // NT, 1, NT)
    edge_attr2 = jnp.pad(edge_attr, ((0, E2 - E), (0, 0))).reshape(EPACK, F)

    data_full = _node_encode(x, ne_W, ne_b)
    sc_agg = _make_sc_aggregate()

    ea0 = _edge_encode(edge_attr2, wb[0], bb[0]).reshape(2, E2, FH)
    agg0 = sc_agg(data_full[0], src, dst, ea0)
    ea1 = _edge_encode(edge_attr2, wb[1], bb[1]).reshape(2, E2, FH)
    agg1 = sc_agg(data_full[1], src, dst, ea1)

    return _node_stage(agg0, agg1, data_full, scale, w1, b1, w2, b2, lng,
                       lnb, batch3, cls_w, cls_b)


# speedup vs baseline: 1.5360x; 1.5360x over previous
import functools

import jax
import jax.numpy as jnp
from jax import lax
from jax.experimental import pallas as pl
from jax.experimental.pallas import tpu as pltpu
from jax.experimental.pallas import tpu_sc as plsc

N = 10000
E = 320000
F = 128
FH = 64
NLAYERS = 2
NG = 64
BN_EPS = 1e-5
LN_EPS = 1e-5

NC, NS = 2, 16
B = 40
EPT = E // NS
NB = EPT // B
MB = 20
NSB = NB // MB
NP = 10240
RPT = NP // NS

NT = 1000
EPACK = E // 8
ETP = 2000


def _data_body(x_ref, w_ref, b_ref, full_ref):
    res = jnp.dot(x_ref[...], w_ref[0], preferred_element_type=jnp.float32)
    full_ref[0] = res + b_ref[0, 0]


def _node_encode(x, ne_W, ne_b):
    return pl.pallas_call(
        _data_body,
        grid=(NLAYERS, N // NT),
        in_specs=[
            pl.BlockSpec((NT, F), lambda l, i: (i, 0)),
            pl.BlockSpec((1, F, F), lambda l, i: (l, 0, 0)),
            pl.BlockSpec((1, 1, F), lambda l, i: (l, 0, 0)),
        ],
        out_specs=pl.BlockSpec((1, NT, F), lambda l, i: (l, i, 0)),
        out_shape=jax.ShapeDtypeStruct((NLAYERS, NP, F), jnp.float32),
    )(x, ne_W, ne_b)


def _ea_body(e_ref, w_ref, b_ref, o_ref):
    o_ref[0] = (
        jnp.dot(e_ref[...], w_ref[0], preferred_element_type=jnp.float32)
        + b_ref[0, 0]
    )


def _edge_encode(edge_attr2, wb, bb):
    return pl.pallas_call(
        _ea_body,
        grid=(2, EPACK // ETP),
        in_specs=[
            pl.BlockSpec((ETP, F), lambda c, i: (i, 0)),
            pl.BlockSpec((1, F, 8 * FH), lambda c, i: (c, 0, 0)),
            pl.BlockSpec((1, 1, 8 * FH), lambda c, i: (c, 0, 0)),
        ],
        out_specs=pl.BlockSpec((1, ETP, 8 * FH), lambda c, i: (c, i, 0)),
        out_shape=jax.ShapeDtypeStruct((2, EPACK, 8 * FH), jnp.float32),
    )(edge_attr2, wb, bb)


def _sc_body(data_hbm, src_hbm, dst_hbm, ea0_hbm, ea1_hbm, out_hbm,
             acc_sh, src2, dst2, dst_w, rows_v0, rows_v1, ea_v0, ea_v1,
             ctr_v, semg, seme):
    c = lax.axis_index("c")
    s = lax.axis_index("s")
    row0 = s * RPT
    e0t = s * EPT
    col0 = c * FH

    def _zb(i, _):
        ctr_v[i // 8, pl.ds((i % 8) * 16, 16)] = jnp.zeros((16,), jnp.float32)
        return _

    for l, ea_hbm in ((0, ea0_hbm), (1, ea1_hbm)):
        lax.fori_loop(0, B * 8, _zb, None)
        for k in range(RPT // B):
            pltpu.sync_copy(ctr_v, acc_sh.at[pl.ds(row0 + k * B, B), :])
        plsc.subcore_barrier()

        def _issue(j, mrow0, rbank, ebank):
            pltpu.async_copy(
                data_hbm.at[l].at[src2.at[pl.ds(j * B, B)]], rbank, semg)
            pltpu.async_copy(
                ea_hbm.at[c, pl.ds(mrow0 + j * B, B), :], ebank, seme)

        def _mb(j, rbank, ebank):
            pltpu.make_async_copy(
                data_hbm.at[l, pl.ds(0, B), :], rbank, semg).wait()
            pltpu.make_async_copy(
                ea_hbm.at[c, pl.ds(0, B), :], ebank, seme).wait()

            @plsc.parallel_loop(0, B, unroll=2)
            def _cb(e):
                for v in range(FH // 16):
                    a = ebank[e, pl.ds(v * 16, 16)]
                    g = rbank[e, pl.ds(col0 + v * 16, 16)]
                    m = jnp.maximum(g + a, 0.0) + 1e-7
                    ex = jnp.exp(m)
                    ctr_v[e, pl.ds(v * 16, 16)] = m * ex
                    ctr_v[e, pl.ds(FH + v * 16, 16)] = ex

            for o in (0, 16, B - 16):
                dst_w[pl.ds(o, 16)] = dst2[pl.ds(j * B + o, 16)]
            pltpu.sync_copy(ctr_v, acc_sh.at[dst_w], add=True)

        def _sb(sb, _):
            mrow0 = e0t + sb * MB * B
            pltpu.sync_copy(src_hbm.at[pl.ds(mrow0, MB * B)], src2)
            pltpu.sync_copy(dst_hbm.at[pl.ds(mrow0, MB * B)], dst2)
            _issue(0, mrow0, rows_v0, ea_v0)

            def _pair(k, _2):
                j0 = 2 * k
                _issue(j0 + 1, mrow0, rows_v1, ea_v1)
                _mb(j0, rows_v0, ea_v0)

                @pl.when(k < MB // 2 - 1)
                def _():
                    _issue(j0 + 2, mrow0, rows_v0, ea_v0)
                _mb(j0 + 1, rows_v1, ea_v1)
                return _2
            lax.fori_loop(0, MB // 2, _pair, None)
            return _
        lax.fori_loop(0, NSB, _sb, None)
        plsc.subcore_barrier()

        for k in range(RPT // B):
            r0 = row0 + k * B
            pltpu.sync_copy(acc_sh.at[pl.ds(r0, B), :], ctr_v)

            def _fb(i, _):
                for v in range(FH // 16):
                    num = ctr_v[i, pl.ds(v * 16, 16)]
                    den = ctr_v[i, pl.ds(FH + v * 16, 16)]
                    ea_v0[i, pl.ds(v * 16, 16)] = num / (den + 1e-16)
                return _
            lax.fori_loop(0, B, _fb, None)
            pltpu.sync_copy(ea_v0, out_hbm.at[l, c, pl.ds(r0, B), :])

        if l + 1 < NLAYERS:
            plsc.subcore_barrier()


@functools.cache
def _make_sc_aggregate():
    return functools.partial(
        pl.kernel,
        out_type=jax.ShapeDtypeStruct((NLAYERS, 2, NP, FH), jnp.float32),
        mesh=plsc.VectorSubcoreMesh(core_axis_name="c", subcore_axis_name="s",
                                    num_cores=NC, num_subcores=NS),
        scratch_types=[
            pltpu.VMEM_SHARED((NP, 2 * FH), jnp.float32),
            pltpu.VMEM((MB * B,), jnp.int32),
            pltpu.VMEM((MB * B,), jnp.int32),
            pltpu.VMEM((B,), jnp.int32),
            pltpu.VMEM((B, F), jnp.float32),
            pltpu.VMEM((B, F), jnp.float32),
            pltpu.VMEM((B, FH), jnp.float32),
            pltpu.VMEM((B, FH), jnp.float32),
            pltpu.VMEM((B, 2 * FH), jnp.float32),
            pltpu.SemaphoreType.DMA,
            pltpu.SemaphoreType.DMA,
        ],
    )(_sc_body)


def _node_body(agg_ref, data_ref, scale_ref, w1_ref, b1_ref,
               w2_ref, b2_ref, lng_ref, lnb_ref, batch_ref,
               cw0_ref, cb0_ref, cw1_ref, cb1_ref, cw2_ref, cb2_ref,
               cw3_ref, cb3_ref, o_ref, pooled):
    i = pl.program_id(0)
    nsteps = pl.num_programs(0)

    @pl.when(i == 0)
    def _():
        pooled[...] = jnp.zeros_like(pooled)

    r = jnp.zeros((NT, F), jnp.float32)
    for l in range(NLAYERS):
        a = jnp.concatenate([agg_ref[l, 0], agg_ref[l, 1]], axis=1)
        d = data_ref[l]
        nrm2 = jnp.sqrt(jnp.sum(a * a, axis=1, keepdims=True))
        msgn = a / jnp.maximum(nrm2, 1e-12)
        xn = jnp.sqrt(jnp.sum(d * d, axis=1, keepdims=True))
        out = msgn * xn * scale_ref[l, 0] + d
        h = jnp.dot(out, w1_ref[l], preferred_element_type=jnp.float32)
        h = jnp.maximum(h + b1_ref[l, 0], 0.0)
        h = jnp.dot(h, w2_ref[l], preferred_element_type=jnp.float32)
        h = h + b2_ref[l, 0]
        mu = jnp.mean(h, axis=1, keepdims=True)
        var = jnp.mean((h - mu) ** 2, axis=1, keepdims=True)
        h = (h - mu) / jnp.sqrt(var + LN_EPS) * lng_ref[l, 0] + lnb_ref[l, 0]
        h = jnp.maximum(h, 0.0)
        hmax = jnp.max(h, axis=1, keepdims=True)
        eh = jnp.exp(h - hmax)
        r = r + eh / jnp.sum(eh, axis=1, keepdims=True)

    bt = batch_ref[0, 0]
    gid = jax.lax.broadcasted_iota(jnp.int32, (NT, NG), 1)
    onehot = jnp.where(bt[:, None] == gid, 1.0, 0.0).astype(jnp.float32)
    pooled[...] += jax.lax.dot_general(
        onehot, r, (((0,), (0,)), ((), ())),
        preferred_element_type=jnp.float32)

    @pl.when(i == nsteps - 1)
    def _():
        g = pooled[...]
        g = jnp.maximum(
            jnp.dot(g, cw0_ref[...], preferred_element_type=jnp.float32)
            + cb0_ref[0], 0.0)
        g = jnp.maximum(
            jnp.dot(g, cw1_ref[...], preferred_element_type=jnp.float32)
            + cb1_ref[0], 0.0)
        g = jnp.maximum(
            jnp.dot(g, cw2_ref[...], preferred_element_type=jnp.float32)
            + cb2_ref[0], 0.0)
        g = jnp.dot(g, cw3_ref[...], preferred_element_type=jnp.float32)
        g = g + cb3_ref[0]
        o_ref[...] = jax.nn.sigmoid(g)


def _node_stage(agg, data_full, scale, w1, b1, w2, b2, lng, lnb,
                batch3, cls_w, cls_b):
    full = lambda shape: pl.BlockSpec(shape, lambda i: tuple(0 for _ in shape))
    return pl.pallas_call(
        _node_body,
        grid=(N // NT,),
        in_specs=[
            pl.BlockSpec((NLAYERS, 2, NT, FH), lambda i: (0, 0, i, 0)),
            pl.BlockSpec((NLAYERS, NT, F), lambda i: (0, i, 0)),
            full((NLAYERS, 1, F)),
            full((NLAYERS, F, 2 * F)),
            full((NLAYERS, 1, 2 * F)),
            full((NLAYERS, 2 * F, F)),
            full((NLAYERS, 1, F)),
            full((NLAYERS, 1, F)),
            full((NLAYERS, 1, F)),
            pl.BlockSpec((1, 1, NT), lambda i: (i, 0, 0)),
            full((F, 2 * F)),
            full((1, 2 * F)),
            full((2 * F, F)),
            full((1, F)),
            full((F, NG)),
            full((1, NG)),
            full((NG, 1)),
            full((1, 1)),
        ],
        out_specs=pl.BlockSpec((NG, 1), lambda i: (0, 0)),
        out_shape=jax.ShapeDtypeStruct((NG, 1), jnp.float32),
        scratch_shapes=[pltpu.VMEM((NG, F), jnp.float32)],
    )(agg, data_full, scale, w1, b1, w2, b2, lng, lnb, batch3,
      *[a for pair in zip(cls_w, cls_b) for a in pair])


def kernel(x, edge_attr, params, edge_index, batch):
    layers = params['layers']
    cls = params['classifier']

    ne_W = jnp.stack([p['ne_W'] for p in layers])
    ne_b = jnp.stack([p['ne_b'] for p in layers])[:, None, :]

    def _blockdiag(p):
        halves = []
        for c2 in range(2):
            h = p['ee_W'][:, c2 * FH:(c2 + 1) * FH]
            halves.append(jax.scipy.linalg.block_diag(*([h] * 8)))
        return jnp.stack(halves)

    wb = [_blockdiag(p) for p in layers]
    bb = [jnp.stack([jnp.tile(p['ee_b'][c2 * FH:(c2 + 1) * FH], 8)[None, :]
                     for c2 in range(2)]) for p in layers]

    sbn = 1.0 / jnp.sqrt(1.0 + BN_EPS)
    w1 = jnp.stack([p['mlp_W1'] * (sbn * p['mlp_bn_g'])[None, :]
                    for p in layers])
    b1 = jnp.stack([(p['mlp_b1'] * sbn * p['mlp_bn_g'] + p['mlp_bn_b'])
                    for p in layers])[:, None, :]
    w2 = jnp.stack([p['mlp_W2'] for p in layers])
    b2 = jnp.stack([p['mlp_b2'] for p in layers])[:, None, :]
    lng = jnp.stack([p['ln_g'] for p in layers])[:, None, :]
    lnb = jnp.stack([p['ln_b'] for p in layers])[:, None, :]
    scale = jnp.stack([jnp.broadcast_to(p['msg_scale'], (F,))
                       for p in layers])[:, None, :]

    cls_w, cls_b = [], []
    cur_s, cur_t = None, None
    for ci, c in enumerate(cls):
        W, b = c['W'], c['b']
        if cur_s is not None:
            W = cur_s[:, None] * W
            b = b + cur_t @ c['W']
        cls_w.append(W)
        cls_b.append(b[None, :])
        if ci < 3:
            cur_s = sbn * c['bn_g']
            cur_t = c['bn_b']

    src = edge_index[0]
    dst = edge_index[1]
    batch3 = batch.reshape(N // NT, 1, NT)
    edge_attr2 = edge_attr.reshape(EPACK, F)

    data_full = _node_encode(x, ne_W, ne_b)
    ea0 = _edge_encode(edge_attr2, wb[0], bb[0]).reshape(2, E, FH)
    ea1 = _edge_encode(edge_attr2, wb[1], bb[1]).reshape(2, E, FH)
    agg = _make_sc_aggregate()(data_full, src, dst, ea0, ea1)

    return _node_stage(agg, data_full, scale, w1, b1, w2, b2, lng,
                       lnb, batch3, cls_w, cls_b)


# speedup vs baseline: 1.7874x; 1.1637x over previous
import functools

import jax
import jax.numpy as jnp
from jax import lax
from jax.experimental import pallas as pl
from jax.experimental.pallas import tpu as pltpu
from jax.experimental.pallas import tpu_sc as plsc

N = 10000
E = 320000
F = 128
FH = 64
NLAYERS = 2
NG = 64
BN_EPS = 1e-5
LN_EPS = 1e-5

NC, NS = 2, 16
B = 40
EPT = E // NS
NB = EPT // B
MB = 20
NSB = NB // MB
NP = 10240
RPT = NP // NS

NT = 1000
EF = 16
ET = 8000


def _data_body(x_ref, w_ref, b_ref, full_ref):
    res = jnp.dot(x_ref[...], w_ref[0], preferred_element_type=jnp.float32)
    full_ref[0] = res + b_ref[0, 0]


def _node_encode(x, ne_W, ne_b):
    return pl.pallas_call(
        _data_body,
        grid=(NLAYERS, N // NT),
        in_specs=[
            pl.BlockSpec((NT, F), lambda l, i: (i, 0)),
            pl.BlockSpec((1, F, F), lambda l, i: (l, 0, 0)),
            pl.BlockSpec((1, 1, F), lambda l, i: (l, 0, 0)),
        ],
        out_specs=pl.BlockSpec((1, NT, F), lambda l, i: (l, i, 0)),
        out_shape=jax.ShapeDtypeStruct((NLAYERS, NP, F), jnp.float32),
    )(x, ne_W, ne_b)


def _ea_body2(e_ref, w_ref, b_ref, o_ref):
    o_ref[0, 0] = (
        jnp.dot(e_ref[...], w_ref[0, 0], preferred_element_type=jnp.float32)
        + b_ref[0, 0, 0]
    )


def _edge_encode(edge_attr, ee_Wh, ee_bh):
    return pl.pallas_call(
        _ea_body2,
        grid=(NLAYERS, 2, E // ET),
        in_specs=[
            pl.BlockSpec((ET, EF), lambda l, c, i: (i, 0)),
            pl.BlockSpec((1, 1, EF, FH), lambda l, c, i: (l, c, 0, 0)),
            pl.BlockSpec((1, 1, 1, FH), lambda l, c, i: (l, c, 0, 0)),
        ],
        out_specs=pl.BlockSpec((1, 1, ET, FH), lambda l, c, i: (l, c, i, 0)),
        out_shape=jax.ShapeDtypeStruct((NLAYERS, 2, E, FH), jnp.float32),
    )(edge_attr, ee_Wh, ee_bh)


def _sc_body(data_hbm, src_hbm, dst_hbm, ea_hbm, out_hbm,
             acc_sh, src2, dst2, dst_w, rows_v0, rows_v1, ea_v0, ea_v1,
             ctr_v, semg, seme):
    c = lax.axis_index("c")
    s = lax.axis_index("s")
    row0 = s * RPT
    e0t = s * EPT
    col0 = c * FH

    def _zb(i, _):
        ctr_v[i // 8, pl.ds((i % 8) * 16, 16)] = jnp.zeros((16,), jnp.float32)
        return _

    for l in range(NLAYERS):
        lax.fori_loop(0, B * 8, _zb, None)
        for k in range(RPT // B):
            pltpu.sync_copy(ctr_v, acc_sh.at[pl.ds(row0 + k * B, B), :])
        plsc.subcore_barrier()

        def _issue(j, mrow0, rbank, ebank):
            pltpu.async_copy(
                data_hbm.at[l].at[src2.at[pl.ds(j * B, B)]], rbank, semg)
            pltpu.async_copy(
                ea_hbm.at[l, c, pl.ds(mrow0 + j * B, B), :], ebank, seme)

        def _mb(j, rbank, ebank):
            pltpu.make_async_copy(
                data_hbm.at[l, pl.ds(0, B), :], rbank, semg).wait()
            pltpu.make_async_copy(
                ea_hbm.at[l, c, pl.ds(0, B), :], ebank, seme).wait()

            @plsc.parallel_loop(0, B, unroll=2)
            def _cb(e):
                for v in range(FH // 16):
                    a = ebank[e, pl.ds(v * 16, 16)]
                    g = rbank[e, pl.ds(col0 + v * 16, 16)]
                    m = jnp.maximum(g + a, 0.0) + 1e-7
                    ex = jnp.exp(m)
                    ctr_v[e, pl.ds(v * 16, 16)] = m * ex
                    ctr_v[e, pl.ds(FH + v * 16, 16)] = ex

            for o in (0, 16, B - 16):
                dst_w[pl.ds(o, 16)] = dst2[pl.ds(j * B + o, 16)]
            pltpu.sync_copy(ctr_v, acc_sh.at[dst_w], add=True)

        def _sb(sb, _):
            mrow0 = e0t + sb * MB * B
            pltpu.sync_copy(src_hbm.at[pl.ds(mrow0, MB * B)], src2)
            pltpu.sync_copy(dst_hbm.at[pl.ds(mrow0, MB * B)], dst2)
            _issue(0, mrow0, rows_v0, ea_v0)

            def _pair(k, _2):
                j0 = 2 * k
                _issue(j0 + 1, mrow0, rows_v1, ea_v1)
                _mb(j0, rows_v0, ea_v0)

                @pl.when(k < MB // 2 - 1)
                def _():
                    _issue(j0 + 2, mrow0, rows_v0, ea_v0)
                _mb(j0 + 1, rows_v1, ea_v1)
                return _2
            lax.fori_loop(0, MB // 2, _pair, None)
            return _
        lax.fori_loop(0, NSB, _sb, None)
        plsc.subcore_barrier()

        for k in range(RPT // B):
            r0 = row0 + k * B
            pltpu.sync_copy(acc_sh.at[pl.ds(r0, B), :], ctr_v)

            def _fb(i, _):
                for v in range(FH // 16):
                    num = ctr_v[i, pl.ds(v * 16, 16)]
                    den = ctr_v[i, pl.ds(FH + v * 16, 16)]
                    ea_v0[i, pl.ds(v * 16, 16)] = num / (den + 1e-16)
                return _
            lax.fori_loop(0, B, _fb, None)
            pltpu.sync_copy(ea_v0, out_hbm.at[l, c, pl.ds(r0, B), :])

        if l + 1 < NLAYERS:
            plsc.subcore_barrier()


@functools.cache
def _make_sc_aggregate():
    return functools.partial(
        pl.kernel,
        out_type=jax.ShapeDtypeStruct((NLAYERS, 2, NP, FH), jnp.float32),
        mesh=plsc.VectorSubcoreMesh(core_axis_name="c", subcore_axis_name="s",
                                    num_cores=NC, num_subcores=NS),
        scratch_types=[
            pltpu.VMEM_SHARED((NP, 2 * FH), jnp.float32),
            pltpu.VMEM((MB * B,), jnp.int32),
            pltpu.VMEM((MB * B,), jnp.int32),
            pltpu.VMEM((B,), jnp.int32),
            pltpu.VMEM((B, F), jnp.float32),
            pltpu.VMEM((B, F), jnp.float32),
            pltpu.VMEM((B, FH), jnp.float32),
            pltpu.VMEM((B, FH), jnp.float32),
            pltpu.VMEM((B, 2 * FH), jnp.float32),
            pltpu.SemaphoreType.DMA,
            pltpu.SemaphoreType.DMA,
        ],
    )(_sc_body)


def _node_body(agg_ref, data_ref, scale_ref, w1_ref, b1_ref,
               w2_ref, b2_ref, lng_ref, lnb_ref, batch_ref,
               cw0_ref, cb0_ref, cw1_ref, cb1_ref, cw2_ref, cb2_ref,
               cw3_ref, cb3_ref, o_ref, pooled):
    i = pl.program_id(0)
    nsteps = pl.num_programs(0)

    @pl.when(i == 0)
    def _():
        pooled[...] = jnp.zeros_like(pooled)

    r = jnp.zeros((NT, F), jnp.float32)
    for l in range(NLAYERS):
        a = jnp.concatenate([agg_ref[l, 0], agg_ref[l, 1]], axis=1)
        d = data_ref[l]
        nrm2 = jnp.sqrt(jnp.sum(a * a, axis=1, keepdims=True))
        msgn = a / jnp.maximum(nrm2, 1e-12)
        xn = jnp.sqrt(jnp.sum(d * d, axis=1, keepdims=True))
        out = msgn * xn * scale_ref[l, 0] + d
        h = jnp.dot(out, w1_ref[l], preferred_element_type=jnp.float32)
        h = jnp.maximum(h + b1_ref[l, 0], 0.0)
        h = jnp.dot(h, w2_ref[l], preferred_element_type=jnp.float32)
        h = h + b2_ref[l, 0]
        mu = jnp.mean(h, axis=1, keepdims=True)
        var = jnp.mean((h - mu) ** 2, axis=1, keepdims=True)
        h = (h - mu) / jnp.sqrt(var + LN_EPS) * lng_ref[l, 0] + lnb_ref[l, 0]
        h = jnp.maximum(h, 0.0)
        hmax = jnp.max(h, axis=1, keepdims=True)
        eh = jnp.exp(h - hmax)
        r = r + eh / jnp.sum(eh, axis=1, keepdims=True)

    bt = batch_ref[0, 0]
    gid = jax.lax.broadcasted_iota(jnp.int32, (NT, NG), 1)
    onehot = jnp.where(bt[:, None] == gid, 1.0, 0.0).astype(jnp.float32)
    pooled[...] += jax.lax.dot_general(
        onehot, r, (((0,), (0,)), ((), ())),
        preferred_element_type=jnp.float32)

    @pl.when(i == nsteps - 1)
    def _():
        g = pooled[...]
        g = jnp.maximum(
            jnp.dot(g, cw0_ref[...], preferred_element_type=jnp.float32)
            + cb0_ref[0], 0.0)
        g = jnp.maximum(
            jnp.dot(g, cw1_ref[...], preferred_element_type=jnp.float32)
            + cb1_ref[0], 0.0)
        g = jnp.maximum(
            jnp.dot(g, cw2_ref[...], preferred_element_type=jnp.float32)
            + cb2_ref[0], 0.0)
        g = jnp.dot(g, cw3_ref[...], preferred_element_type=jnp.float32)
        g = g + cb3_ref[0]
        o_ref[...] = jax.nn.sigmoid(g)


def _node_stage(agg, data_full, scale, w1, b1, w2, b2, lng, lnb,
                batch3, cls_w, cls_b):
    full = lambda shape: pl.BlockSpec(shape, lambda i: tuple(0 for _ in shape))
    return pl.pallas_call(
        _node_body,
        grid=(N // NT,),
        in_specs=[
            pl.BlockSpec((NLAYERS, 2, NT, FH), lambda i: (0, 0, i, 0)),
            pl.BlockSpec((NLAYERS, NT, F), lambda i: (0, i, 0)),
            full((NLAYERS, 1, F)),
            full((NLAYERS, F, 2 * F)),
            full((NLAYERS, 1, 2 * F)),
            full((NLAYERS, 2 * F, F)),
            full((NLAYERS, 1, F)),
            full((NLAYERS, 1, F)),
            full((NLAYERS, 1, F)),
            pl.BlockSpec((1, 1, NT), lambda i: (i, 0, 0)),
            full((F, 2 * F)),
            full((1, 2 * F)),
            full((2 * F, F)),
            full((1, F)),
            full((F, NG)),
            full((1, NG)),
            full((NG, 1)),
            full((1, 1)),
        ],
        out_specs=pl.BlockSpec((NG, 1), lambda i: (0, 0)),
        out_shape=jax.ShapeDtypeStruct((NG, 1), jnp.float32),
        scratch_shapes=[pltpu.VMEM((NG, F), jnp.float32)],
    )(agg, data_full, scale, w1, b1, w2, b2, lng, lnb, batch3,
      *[a for pair in zip(cls_w, cls_b) for a in pair])


def kernel(x, edge_attr, params, edge_index, batch):
    layers = params['layers']
    cls = params['classifier']

    ne_W = jnp.stack([p['ne_W'] for p in layers])
    ne_b = jnp.stack([p['ne_b'] for p in layers])[:, None, :]

    ee_Wh = jnp.stack(
        [jnp.stack([p['ee_W'][:, :FH], p['ee_W'][:, FH:]]) for p in layers])
    ee_bh = jnp.stack(
        [jnp.stack([p['ee_b'][None, :FH], p['ee_b'][None, FH:]])
         for p in layers])

    sbn = 1.0 / jnp.sqrt(1.0 + BN_EPS)
    w1 = jnp.stack([p['mlp_W1'] * (sbn * p['mlp_bn_g'])[None, :]
                    for p in layers])
    b1 = jnp.stack([(p['mlp_b1'] * sbn * p['mlp_bn_g'] + p['mlp_bn_b'])
                    for p in layers])[:, None, :]
    w2 = jnp.stack([p['mlp_W2'] for p in layers])
    b2 = jnp.stack([p['mlp_b2'] for p in layers])[:, None, :]
    lng = jnp.stack([p['ln_g'] for p in layers])[:, None, :]
    lnb = jnp.stack([p['ln_b'] for p in layers])[:, None, :]
    scale = jnp.stack([jnp.broadcast_to(p['msg_scale'], (F,))
                       for p in layers])[:, None, :]

    cls_w, cls_b = [], []
    cur_s, cur_t = None, None
    for ci, c in enumerate(cls):
        W, b = c['W'], c['b']
        if cur_s is not None:
            W = cur_s[:, None] * W
            b = b + cur_t @ c['W']
        cls_w.append(W)
        cls_b.append(b[None, :])
        if ci < 3:
            cur_s = sbn * c['bn_g']
            cur_t = c['bn_b']

    src = edge_index[0]
    dst = edge_index[1]
    batch3 = batch.reshape(N // NT, 1, NT)

    data_full = _node_encode(x, ne_W, ne_b)
    ea = _edge_encode(edge_attr, ee_Wh, ee_bh)
    agg = _make_sc_aggregate()(data_full, src, dst, ea)

    return _node_stage(agg, data_full, scale, w1, b1, w2, b2, lng,
                       lnb, batch3, cls_w, cls_b)


# speedup vs baseline: 1.8563x; 1.0386x over previous
import functools

import jax
import jax.numpy as jnp
from jax import lax
from jax.experimental import pallas as pl
from jax.experimental.pallas import tpu as pltpu
from jax.experimental.pallas import tpu_sc as plsc

N = 10000
E = 320000
F = 128
FH = 64
NLAYERS = 2
NG = 64
BN_EPS = 1e-5
LN_EPS = 1e-5

NC, NS = 2, 16
B = 40
EPT = E // NS
NB = EPT // B
MB = 50
NSB = NB // MB
NP = 10240
RPT = NP // NS

NT = 1000
EF = 16
ET = 8000


def _data_body(x_ref, w_ref, b_ref, full_ref):
    res = jnp.dot(x_ref[...], w_ref[0], preferred_element_type=jnp.float32)
    full_ref[0] = res + b_ref[0, 0]


def _node_encode(x, ne_W, ne_b):
    return pl.pallas_call(
        _data_body,
        grid=(NLAYERS, N // NT),
        in_specs=[
            pl.BlockSpec((NT, F), lambda l, i: (i, 0)),
            pl.BlockSpec((1, F, F), lambda l, i: (l, 0, 0)),
            pl.BlockSpec((1, 1, F), lambda l, i: (l, 0, 0)),
        ],
        out_specs=pl.BlockSpec((1, NT, F), lambda l, i: (l, i, 0)),
        out_shape=jax.ShapeDtypeStruct((NLAYERS, NP, F), jnp.float32),
    )(x, ne_W, ne_b)


def _ea_body2(e_ref, w_ref, b_ref, o_ref):
    o_ref[0, 0] = (
        jnp.dot(e_ref[...], w_ref[0, 0], preferred_element_type=jnp.float32)
        + b_ref[0, 0, 0]
    )


def _edge_encode(edge_attr, ee_Wh, ee_bh):
    return pl.pallas_call(
        _ea_body2,
        grid=(NLAYERS, 2, E // ET),
        in_specs=[
            pl.BlockSpec((ET, EF), lambda l, c, i: (i, 0)),
            pl.BlockSpec((1, 1, EF, FH), lambda l, c, i: (l, c, 0, 0)),
            pl.BlockSpec((1, 1, 1, FH), lambda l, c, i: (l, c, 0, 0)),
        ],
        out_specs=pl.BlockSpec((1, 1, ET, FH), lambda l, c, i: (l, c, i, 0)),
        out_shape=jax.ShapeDtypeStruct((NLAYERS, 2, E, FH), jnp.float32),
    )(edge_attr, ee_Wh, ee_bh)


def _sc_body(data_hbm, src_hbm, dst_hbm, ea_hbm, out_hbm,
             acc_sh, src2, dst2, dst_w, rows_v0, rows_v1, ea_v0, ea_v1,
             ctr_v, semg, seme):
    c = lax.axis_index("c")
    s = lax.axis_index("s")
    row0 = s * RPT
    e0t = s * EPT
    col0 = c * FH

    def _zb(i, _):
        ctr_v[i // 8, pl.ds((i % 8) * 16, 16)] = jnp.zeros((16,), jnp.float32)
        return _

    for l in range(NLAYERS):
        lax.fori_loop(0, B * 8, _zb, None)
        for k in range(RPT // B):
            pltpu.sync_copy(ctr_v, acc_sh.at[pl.ds(row0 + k * B, B), :])
        plsc.subcore_barrier()

        def _issue(j, mrow0, rbank, ebank):
            pltpu.async_copy(
                data_hbm.at[l].at[src2.at[pl.ds(j * B, B)]], rbank, semg)
            pltpu.async_copy(
                ea_hbm.at[l, c, pl.ds(mrow0 + j * B, B), :], ebank, seme)

        def _mb(j, rbank, ebank):
            pltpu.make_async_copy(
                data_hbm.at[l, pl.ds(0, B), :], rbank, semg).wait()
            pltpu.make_async_copy(
                ea_hbm.at[l, c, pl.ds(0, B), :], ebank, seme).wait()

            @plsc.parallel_loop(0, B, unroll=2)
            def _cb(e):
                for v in range(FH // 16):
                    a = ebank[e, pl.ds(v * 16, 16)]
                    g = rbank[e, pl.ds(col0 + v * 16, 16)]
                    m = jnp.maximum(g + a, 0.0) + 1e-7
                    ex = jnp.exp(m)
                    ctr_v[e, pl.ds(v * 16, 16)] = m * ex
                    ctr_v[e, pl.ds(FH + v * 16, 16)] = ex

            for o in (0, 16, B - 16):
                dst_w[pl.ds(o, 16)] = dst2[pl.ds(j * B + o, 16)]
            pltpu.sync_copy(ctr_v, acc_sh.at[dst_w], add=True)

        def _sb(sb, _):
            mrow0 = e0t + sb * MB * B
            pltpu.sync_copy(src_hbm.at[pl.ds(mrow0, MB * B)], src2)
            pltpu.sync_copy(dst_hbm.at[pl.ds(mrow0, MB * B)], dst2)
            _issue(0, mrow0, rows_v0, ea_v0)

            def _pair(k, _2):
                j0 = 2 * k
                _issue(j0 + 1, mrow0, rows_v1, ea_v1)
                _mb(j0, rows_v0, ea_v0)

                @pl.when(k < MB // 2 - 1)
                def _():
                    _issue(j0 + 2, mrow0, rows_v0, ea_v0)
                _mb(j0 + 1, rows_v1, ea_v1)
                return _2
            lax.fori_loop(0, MB // 2, _pair, None)
            return _
        lax.fori_loop(0, NSB, _sb, None)
        plsc.subcore_barrier()

        for k in range(RPT // B):
            r0 = row0 + k * B
            pltpu.sync_copy(acc_sh.at[pl.ds(r0, B), :], ctr_v)

            def _fb(i, _):
                for v in range(FH // 16):
                    num = ctr_v[i, pl.ds(v * 16, 16)]
                    den = ctr_v[i, pl.ds(FH + v * 16, 16)]
                    ea_v0[i, pl.ds(v * 16, 16)] = num / (den + 1e-16)
                return _
            lax.fori_loop(0, B, _fb, None)
            pltpu.sync_copy(ea_v0, out_hbm.at[l, c, pl.ds(r0, B), :])

        if l + 1 < NLAYERS:
            plsc.subcore_barrier()


@functools.cache
def _make_sc_aggregate():
    return functools.partial(
        pl.kernel,
        out_type=jax.ShapeDtypeStruct((NLAYERS, 2, NP, FH), jnp.float32),
        mesh=plsc.VectorSubcoreMesh(core_axis_name="c", subcore_axis_name="s",
                                    num_cores=NC, num_subcores=NS),
        scratch_types=[
            pltpu.VMEM_SHARED((NP, 2 * FH), jnp.float32),
            pltpu.VMEM((MB * B,), jnp.int32),
            pltpu.VMEM((MB * B,), jnp.int32),
            pltpu.VMEM((B,), jnp.int32),
            pltpu.VMEM((B, F), jnp.float32),
            pltpu.VMEM((B, F), jnp.float32),
            pltpu.VMEM((B, FH), jnp.float32),
            pltpu.VMEM((B, FH), jnp.float32),
            pltpu.VMEM((B, 2 * FH), jnp.float32),
            pltpu.SemaphoreType.DMA,
            pltpu.SemaphoreType.DMA,
        ],
    )(_sc_body)


def _node_body(agg_ref, data_ref, scale_ref, w1_ref, b1_ref,
               w2_ref, b2_ref, lng_ref, lnb_ref, batch_ref,
               cw0_ref, cb0_ref, cw1_ref, cb1_ref, cw2_ref, cb2_ref,
               cw3_ref, cb3_ref, o_ref, pooled):
    i = pl.program_id(0)
    nsteps = pl.num_programs(0)

    @pl.when(i == 0)
    def _():
        pooled[...] = jnp.zeros_like(pooled)

    r = jnp.zeros((NT, F), jnp.float32)
    for l in range(NLAYERS):
        a = jnp.concatenate([agg_ref[l, 0], agg_ref[l, 1]], axis=1)
        d = data_ref[l]
        nrm2 = jnp.sqrt(jnp.sum(a * a, axis=1, keepdims=True))
        msgn = a / jnp.maximum(nrm2, 1e-12)
        xn = jnp.sqrt(jnp.sum(d * d, axis=1, keepdims=True))
        out = msgn * xn * scale_ref[l, 0] + d
        h = jnp.dot(out, w1_ref[l], preferred_element_type=jnp.float32)
        h = jnp.maximum(h + b1_ref[l, 0], 0.0)
        h = jnp.dot(h, w2_ref[l], preferred_element_type=jnp.float32)
        h = h + b2_ref[l, 0]
        mu = jnp.mean(h, axis=1, keepdims=True)
        var = jnp.mean((h - mu) ** 2, axis=1, keepdims=True)
        h = (h - mu) / jnp.sqrt(var + LN_EPS) * lng_ref[l, 0] + lnb_ref[l, 0]
        h = jnp.maximum(h, 0.0)
        hmax = jnp.max(h, axis=1, keepdims=True)
        eh = jnp.exp(h - hmax)
        r = r + eh / jnp.sum(eh, axis=1, keepdims=True)

    bt = batch_ref[0, 0]
    gid = jax.lax.broadcasted_iota(jnp.int32, (NT, NG), 1)
    onehot = jnp.where(bt[:, None] == gid, 1.0, 0.0).astype(jnp.float32)
    pooled[...] += jax.lax.dot_general(
        onehot, r, (((0,), (0,)), ((), ())),
        preferred_element_type=jnp.float32)

    @pl.when(i == nsteps - 1)
    def _():
        g = pooled[...]
        g = jnp.maximum(
            jnp.dot(g, cw0_ref[...], preferred_element_type=jnp.float32)
            + cb0_ref[0], 0.0)
        g = jnp.maximum(
            jnp.dot(g, cw1_ref[...], preferred_element_type=jnp.float32)
            + cb1_ref[0], 0.0)
        g = jnp.maximum(
            jnp.dot(g, cw2_ref[...], preferred_element_type=jnp.float32)
            + cb2_ref[0], 0.0)
        g = jnp.dot(g, cw3_ref[...], preferred_element_type=jnp.float32)
        g = g + cb3_ref[0]
        o_ref[...] = jax.nn.sigmoid(g)


def _node_stage(agg, data_full, scale, w1, b1, w2, b2, lng, lnb,
                batch3, cls_w, cls_b):
    full = lambda shape: pl.BlockSpec(shape, lambda i: tuple(0 for _ in shape))
    return pl.pallas_call(
        _node_body,
        grid=(N // NT,),
        in_specs=[
            pl.BlockSpec((NLAYERS, 2, NT, FH), lambda i: (0, 0, i, 0)),
            pl.BlockSpec((NLAYERS, NT, F), lambda i: (0, i, 0)),
            full((NLAYERS, 1, F)),
            full((NLAYERS, F, 2 * F)),
            full((NLAYERS, 1, 2 * F)),
            full((NLAYERS, 2 * F, F)),
            full((NLAYERS, 1, F)),
            full((NLAYERS, 1, F)),
            full((NLAYERS, 1, F)),
            pl.BlockSpec((1, 1, NT), lambda i: (i, 0, 0)),
            full((F, 2 * F)),
            full((1, 2 * F)),
            full((2 * F, F)),
            full((1, F)),
            full((F, NG)),
            full((1, NG)),
            full((NG, 1)),
            full((1, 1)),
        ],
        out_specs=pl.BlockSpec((NG, 1), lambda i: (0, 0)),
        out_shape=jax.ShapeDtypeStruct((NG, 1), jnp.float32),
        scratch_shapes=[pltpu.VMEM((NG, F), jnp.float32)],
    )(agg, data_full, scale, w1, b1, w2, b2, lng, lnb, batch3,
      *[a for pair in zip(cls_w, cls_b) for a in pair])


def kernel(x, edge_attr, params, edge_index, batch):
    layers = params['layers']
    cls = params['classifier']

    ne_W = jnp.stack([p['ne_W'] for p in layers])
    ne_b = jnp.stack([p['ne_b'] for p in layers])[:, None, :]

    ee_Wh = jnp.stack(
        [jnp.stack([p['ee_W'][:, :FH], p['ee_W'][:, FH:]]) for p in layers])
    ee_bh = jnp.stack(
        [jnp.stack([p['ee_b'][None, :FH], p['ee_b'][None, FH:]])
         for p in layers])

    sbn = 1.0 / jnp.sqrt(1.0 + BN_EPS)
    w1 = jnp.stack([p['mlp_W1'] * (sbn * p['mlp_bn_g'])[None, :]
                    for p in layers])
    b1 = jnp.stack([(p['mlp_b1'] * sbn * p['mlp_bn_g'] + p['mlp_bn_b'])
                    for p in layers])[:, None, :]
    w2 = jnp.stack([p['mlp_W2'] for p in layers])
    b2 = jnp.stack([p['mlp_b2'] for p in layers])[:, None, :]
    lng = jnp.stack([p['ln_g'] for p in layers])[:, None, :]
    lnb = jnp.stack([p['ln_b'] for p in layers])[:, None, :]
    scale = jnp.stack([jnp.broadcast_to(p['msg_scale'], (F,))
                       for p in layers])[:, None, :]

    cls_w, cls_b = [], []
    cur_s, cur_t = None, None
    for ci, c in enumerate(cls):
        W, b = c['W'], c['b']
        if cur_s is not None:
            W = cur_s[:, None] * W
            b = b + cur_t @ c['W']
        cls_w.append(W)
        cls_b.append(b[None, :])
        if ci < 3:
            cur_s = sbn * c['bn_g']
            cur_t = c['bn_b']

    src = edge_index[0]
    dst = edge_index[1]
    batch3 = batch.reshape(N // NT, 1, NT)

    data_full = _node_encode(x, ne_W, ne_b)
    ea = _edge_encode(edge_attr, ee_Wh, ee_bh)
    agg = _make_sc_aggregate()(data_full, src, dst, ea)

    return _node_stage(agg, data_full, scale, w1, b1, w2, b2, lng,
                       lnb, batch3, cls_w, cls_b)


# speedup vs baseline: 1.8665x; 1.0055x over previous
import functools

import jax
import jax.numpy as jnp
from jax import lax
from jax.experimental import pallas as pl
from jax.experimental.pallas import tpu as pltpu
from jax.experimental.pallas import tpu_sc as plsc

N = 10000
E = 320000
F = 128
FH = 64
NLAYERS = 2
NG = 64
BN_EPS = 1e-5
LN_EPS = 1e-5

NC, NS = 2, 16
B = 40
EPT = E // NS
NB = EPT // B
MB = 50
NSB = NB // MB
NP = 10240
RPT = NP // NS

NT = 1000
EF = 16
ET = 8000


def _data_body(x_ref, w_ref, b_ref, full_ref):
    res = jnp.dot(x_ref[...], w_ref[0], preferred_element_type=jnp.float32)
    full_ref[0] = res + b_ref[0, 0]


def _node_encode(x, ne_W, ne_b):
    return pl.pallas_call(
        _data_body,
        grid=(NLAYERS, N // NT),
        in_specs=[
            pl.BlockSpec((NT, F), lambda l, i: (i, 0)),
            pl.BlockSpec((1, F, F), lambda l, i: (l, 0, 0)),
            pl.BlockSpec((1, 1, F), lambda l, i: (l, 0, 0)),
        ],
        out_specs=pl.BlockSpec((1, NT, F), lambda l, i: (l, i, 0)),
        out_shape=jax.ShapeDtypeStruct((NLAYERS, NP, F), jnp.float32),
    )(x, ne_W, ne_b)


def _ea_body2(e_ref, w_ref, b_ref, o_ref):
    o_ref[0, 0] = (
        jnp.dot(e_ref[...], w_ref[0, 0], preferred_element_type=jnp.float32)
        + b_ref[0, 0, 0]
    )


def _edge_encode(edge_attr, ee_Wh, ee_bh):
    return pl.pallas_call(
        _ea_body2,
        grid=(NLAYERS, 2, E // ET),
        in_specs=[
            pl.BlockSpec((ET, EF), lambda l, c, i: (i, 0)),
            pl.BlockSpec((1, 1, EF, FH), lambda l, c, i: (l, c, 0, 0)),
            pl.BlockSpec((1, 1, 1, FH), lambda l, c, i: (l, c, 0, 0)),
        ],
        out_specs=pl.BlockSpec((1, 1, ET, FH), lambda l, c, i: (l, c, i, 0)),
        out_shape=jax.ShapeDtypeStruct((NLAYERS, 2, E, FH), jnp.float32),
    )(edge_attr, ee_Wh, ee_bh)


def _sc_body(data_hbm, src_hbm, dst_hbm, ea_hbm, out_hbm,
             acc_sh, src2, dst2, dst_w, rows_v0, rows_v1, ea_v0, ea_v1,
             ctr_v, semg, seme):
    c = lax.axis_index("c")
    s = lax.axis_index("s")
    row0 = s * RPT
    e0t = s * EPT
    col0 = c * FH

    def _zb(i, _):
        ctr_v[i // 8, pl.ds((i % 8) * 16, 16)] = jnp.zeros((16,), jnp.float32)
        return _

    for l in range(NLAYERS):
        lax.fori_loop(0, B * 8, _zb, None)
        for k in range(RPT // B):
            pltpu.sync_copy(ctr_v, acc_sh.at[pl.ds(row0 + k * B, B), :])
        plsc.subcore_barrier()

        def _issue(j, mrow0, rbank, ebank):
            pltpu.async_copy(
                data_hbm.at[l].at[src2.at[pl.ds(j * B, B)]], rbank, semg)
            pltpu.async_copy(
                ea_hbm.at[l, c, pl.ds(mrow0 + j * B, B), :], ebank, seme)

        def _mb(j, rbank, ebank):
            pltpu.make_async_copy(
                data_hbm.at[l, pl.ds(0, B), :], rbank, semg).wait()
            pltpu.make_async_copy(
                ea_hbm.at[l, c, pl.ds(0, B), :], ebank, seme).wait()

            @plsc.parallel_loop(0, B, unroll=4)
            def _cb(e):
                for v in range(FH // 16):
                    a = ebank[e, pl.ds(v * 16, 16)]
                    g = rbank[e, pl.ds(col0 + v * 16, 16)]
                    m = jnp.maximum(g + a, 0.0) + 1e-7
                    ex = jnp.exp(m)
                    ctr_v[e, pl.ds(v * 16, 16)] = m * ex
                    ctr_v[e, pl.ds(FH + v * 16, 16)] = ex

            for o in (0, 16, B - 16):
                dst_w[pl.ds(o, 16)] = dst2[pl.ds(j * B + o, 16)]
            pltpu.sync_copy(ctr_v, acc_sh.at[dst_w], add=True)

        def _sb(sb, _):
            mrow0 = e0t + sb * MB * B
            pltpu.sync_copy(src_hbm.at[pl.ds(mrow0, MB * B)], src2)
            pltpu.sync_copy(dst_hbm.at[pl.ds(mrow0, MB * B)], dst2)
            _issue(0, mrow0, rows_v0, ea_v0)

            def _pair(k, _2):
                j0 = 2 * k
                _issue(j0 + 1, mrow0, rows_v1, ea_v1)
                _mb(j0, rows_v0, ea_v0)

                @pl.when(k < MB // 2 - 1)
                def _():
                    _issue(j0 + 2, mrow0, rows_v0, ea_v0)
                _mb(j0 + 1, rows_v1, ea_v1)
                return _2
            lax.fori_loop(0, MB // 2, _pair, None)
            return _
        lax.fori_loop(0, NSB, _sb, None)
        plsc.subcore_barrier()

        for k in range(RPT // B):
            r0 = row0 + k * B
            pltpu.sync_copy(acc_sh.at[pl.ds(r0, B), :], ctr_v)

            def _fb(i, _):
                for v in range(FH // 16):
                    num = ctr_v[i, pl.ds(v * 16, 16)]
                    den = ctr_v[i, pl.ds(FH + v * 16, 16)]
                    ea_v0[i, pl.ds(v * 16, 16)] = num / (den + 1e-16)
                return _
            lax.fori_loop(0, B, _fb, None)
            pltpu.sync_copy(ea_v0, out_hbm.at[l, c, pl.ds(r0, B), :])

        if l + 1 < NLAYERS:
            plsc.subcore_barrier()


@functools.cache
def _make_sc_aggregate():
    return functools.partial(
        pl.kernel,
        out_type=jax.ShapeDtypeStruct((NLAYERS, 2, NP, FH), jnp.float32),
        mesh=plsc.VectorSubcoreMesh(core_axis_name="c", subcore_axis_name="s",
                                    num_cores=NC, num_subcores=NS),
        scratch_types=[
            pltpu.VMEM_SHARED((NP, 2 * FH), jnp.float32),
            pltpu.VMEM((MB * B,), jnp.int32),
            pltpu.VMEM((MB * B,), jnp.int32),
            pltpu.VMEM((B,), jnp.int32),
            pltpu.VMEM((B, F), jnp.float32),
            pltpu.VMEM((B, F), jnp.float32),
            pltpu.VMEM((B, FH), jnp.float32),
            pltpu.VMEM((B, FH), jnp.float32),
            pltpu.VMEM((B, 2 * FH), jnp.float32),
            pltpu.SemaphoreType.DMA,
            pltpu.SemaphoreType.DMA,
        ],
    )(_sc_body)


def _node_body(agg_ref, data_ref, scale_ref, w1_ref, b1_ref,
               w2_ref, b2_ref, lng_ref, lnb_ref, batch_ref,
               cw0_ref, cb0_ref, cw1_ref, cb1_ref, cw2_ref, cb2_ref,
               cw3_ref, cb3_ref, o_ref, pooled):
    i = pl.program_id(0)
    nsteps = pl.num_programs(0)

    @pl.when(i == 0)
    def _():
        pooled[...] = jnp.zeros_like(pooled)

    r = jnp.zeros((NT, F), jnp.float32)
    for l in range(NLAYERS):
        a = jnp.concatenate([agg_ref[l, 0], agg_ref[l, 1]], axis=1)
        d = data_ref[l]
        nrm2 = jnp.sqrt(jnp.sum(a * a, axis=1, keepdims=True))
        msgn = a / jnp.maximum(nrm2, 1e-12)
        xn = jnp.sqrt(jnp.sum(d * d, axis=1, keepdims=True))
        out = msgn * xn * scale_ref[l, 0] + d
        h = jnp.dot(out, w1_ref[l], preferred_element_type=jnp.float32)
        h = jnp.maximum(h + b1_ref[l, 0], 0.0)
        h = jnp.dot(h, w2_ref[l], preferred_element_type=jnp.float32)
        h = h + b2_ref[l, 0]
        mu = jnp.mean(h, axis=1, keepdims=True)
        var = jnp.mean((h - mu) ** 2, axis=1, keepdims=True)
        h = (h - mu) / jnp.sqrt(var + LN_EPS) * lng_ref[l, 0] + lnb_ref[l, 0]
        h = jnp.maximum(h, 0.0)
        hmax = jnp.max(h, axis=1, keepdims=True)
        eh = jnp.exp(h - hmax)
        r = r + eh / jnp.sum(eh, axis=1, keepdims=True)

    bt = batch_ref[0, 0]
    gid = jax.lax.broadcasted_iota(jnp.int32, (NT, NG), 1)
    onehot = jnp.where(bt[:, None] == gid, 1.0, 0.0).astype(jnp.float32)
    pooled[...] += jax.lax.dot_general(
        onehot, r, (((0,), (0,)), ((), ())),
        preferred_element_type=jnp.float32)

    @pl.when(i == nsteps - 1)
    def _():
        g = pooled[...]
        g = jnp.maximum(
            jnp.dot(g, cw0_ref[...], preferred_element_type=jnp.float32)
            + cb0_ref[0], 0.0)
        g = jnp.maximum(
            jnp.dot(g, cw1_ref[...], preferred_element_type=jnp.float32)
            + cb1_ref[0], 0.0)
        g = jnp.maximum(
            jnp.dot(g, cw2_ref[...], preferred_element_type=jnp.float32)
            + cb2_ref[0], 0.0)
        g = jnp.dot(g, cw3_ref[...], preferred_element_type=jnp.float32)
        g = g + cb3_ref[0]
        o_ref[...] = jax.nn.sigmoid(g)


def _node_stage(agg, data_full, scale, w1, b1, w2, b2, lng, lnb,
                batch3, cls_w, cls_b):
    full = lambda shape: pl.BlockSpec(shape, lambda i: tuple(0 for _ in shape))
    return pl.pallas_call(
        _node_body,
        grid=(N // NT,),
        in_specs=[
            pl.BlockSpec((NLAYERS, 2, NT, FH), lambda i: (0, 0, i, 0)),
            pl.BlockSpec((NLAYERS, NT, F), lambda i: (0, i, 0)),
            full((NLAYERS, 1, F)),
            full((NLAYERS, F, 2 * F)),
            full((NLAYERS, 1, 2 * F)),
            full((NLAYERS, 2 * F, F)),
            full((NLAYERS, 1, F)),
            full((NLAYERS, 1, F)),
            full((NLAYERS, 1, F)),
            pl.BlockSpec((1, 1, NT), lambda i: (i, 0, 0)),
            full((F, 2 * F)),
            full((1, 2 * F)),
            full((2 * F, F)),
            full((1, F)),
            full((F, NG)),
            full((1, NG)),
            full((NG, 1)),
            full((1, 1)),
        ],
        out_specs=pl.BlockSpec((NG, 1), lambda i: (0, 0)),
        out_shape=jax.ShapeDtypeStruct((NG, 1), jnp.float32),
        scratch_shapes=[pltpu.VMEM((NG, F), jnp.float32)],
    )(agg, data_full, scale, w1, b1, w2, b2, lng, lnb, batch3,
      *[a for pair in zip(cls_w, cls_b) for a in pair])


def kernel(x, edge_attr, params, edge_index, batch):
    layers = params['layers']
    cls = params['classifier']

    ne_W = jnp.stack([p['ne_W'] for p in layers])
    ne_b = jnp.stack([p['ne_b'] for p in layers])[:, None, :]

    ee_Wh = jnp.stack(
        [jnp.stack([p['ee_W'][:, :FH], p['ee_W'][:, FH:]]) for p in layers])
    ee_bh = jnp.stack(
        [jnp.stack([p['ee_b'][None, :FH], p['ee_b'][None, FH:]])
         for p in layers])

    sbn = 1.0 / jnp.sqrt(1.0 + BN_EPS)
    w1 = jnp.stack([p['mlp_W1'] * (sbn * p['mlp_bn_g'])[None, :]
                    for p in layers])
    b1 = jnp.stack([(p['mlp_b1'] * sbn * p['mlp_bn_g'] + p['mlp_bn_b'])
                    for p in layers])[:, None, :]
    w2 = jnp.stack([p['mlp_W2'] for p in layers])
    b2 = jnp.stack([p['mlp_b2'] for p in layers])[:, None, :]
    lng = jnp.stack([p['ln_g'] for p in layers])[:, None, :]
    lnb = jnp.stack([p['ln_b'] for p in layers])[:, None, :]
    scale = jnp.stack([jnp.broadcast_to(p['msg_scale'], (F,))
                       for p in layers])[:, None, :]

    cls_w, cls_b = [], []
    cur_s, cur_t = None, None
    for ci, c in enumerate(cls):
        W, b = c['W'], c['b']
        if cur_s is not None:
            W = cur_s[:, None] * W
            b = b + cur_t @ c['W']
        cls_w.append(W)
        cls_b.append(b[None, :])
        if ci < 3:
            cur_s = sbn * c['bn_g']
            cur_t = c['bn_b']

    src = edge_index[0]
    dst = edge_index[1]
    batch3 = batch.reshape(N // NT, 1, NT)

    data_full = _node_encode(x, ne_W, ne_b)
    ea = _edge_encode(edge_attr, ee_Wh, ee_bh)
    agg = _make_sc_aggregate()(data_full, src, dst, ea)

    return _node_stage(agg, data_full, scale, w1, b1, w2, b2, lng,
                       lnb, batch3, cls_w, cls_b)
